# baseline (speedup 1.0000x reference)
import jax
import jax.numpy as jnp
from jax import lax
from jax.experimental import pallas as pl
from jax.experimental.pallas import tpu as pltpu
from jax.experimental.pallas import tpu_sc as plsc

N = 10000
E = 320000
F_IN = 128
OUT = 32
T = 12
G = 6
TPG = T // G
CG = 2 * OUT * TPG
HCG = CG // 2
NC = 2
NS = 16
NW = NC * NS
GPC = G // NC
EPT_A = E // NW
EPT_C = E // NS
SCE = 2000
NSC = EPT_C // SCE
K = 80
NBURST = SCE // K
NP = 10240
ROWS_PT = NP // NS
BN = 400
NB = N // BN

_MESH = plsc.VectorSubcoreMesh(
    core_axis_name="c", subcore_axis_name="s", num_cores=NC, num_subcores=NS)


def _deg_body(ei_hbm, ew_hbm, out_hbm, dst_v, ew_v, hist_v):
    c = lax.axis_index("c")
    s = lax.axis_index("s")
    wid = c * NS + s

    def zero_hist(i, _):
        hist_v[pl.ds(i * 16, 16)] = jnp.zeros((16,), jnp.float32)
        return 0

    lax.fori_loop(0, N // 16, zero_hist, 0)

    pltpu.sync_copy(ei_hbm.at[pl.ds(E + wid * EPT_A, EPT_A)], dst_v)
    pltpu.sync_copy(ew_hbm.at[pl.ds(wid * EPT_A, EPT_A)], ew_v)

    def accum(i, _):
        idx = dst_v[pl.ds(i * 16, 16)]
        w = ew_v[pl.ds(i * 16, 16)]
        plsc.addupdate_scatter(hist_v, [idx], w)
        return 0

    lax.fori_loop(0, EPT_A // 16, accum, 0)

    pltpu.sync_copy(hist_v, out_hbm.at[wid])


_deg_call = pl.kernel(
    _deg_body,
    out_type=jax.ShapeDtypeStruct((NW, N), jnp.float32),
    mesh=_MESH,
    compiler_params=pltpu.CompilerParams(needs_layout_passes=False),
    scratch_types=[
        pltpu.VMEM((EPT_A,), jnp.int32),
        pltpu.VMEM((EPT_A,), jnp.float32),
        pltpu.VMEM((N,), jnp.float32),
    ],
)


def _dense_body(x_ref, w_ref, wp_ref, deg_ref, out_ref, outbf_ref):
    xb = x_ref[...]
    dd = deg_ref[...]
    dinv = lax.rsqrt(1.0 + jnp.sum(dd, axis=1, keepdims=True))
    acts = jnp.dot(xb, w_ref[0], preferred_element_type=jnp.float32) * dinv
    out_ref[...] = acts
    actsp = jnp.dot(xb, wp_ref[0], preferred_element_type=jnp.float32) * dinv
    outbf_ref[...] = actsp.astype(jnp.bfloat16)


def _dense_call(xf, wblocks, wperm, degT):
    return pl.pallas_call(
        _dense_body,
        grid=(NB, G),
        in_specs=[
            pl.BlockSpec((BN, F_IN * T), lambda i, g: (i, 0)),
            pl.BlockSpec((1, F_IN * T, CG), lambda i, g: (g, 0, 0)),
            pl.BlockSpec((1, F_IN * T, CG), lambda i, g: (g, 0, 0)),
            pl.BlockSpec((BN, NW), lambda i, g: (i, 0)),
        ],
        out_specs=[
            pl.BlockSpec((BN, CG), lambda i, g: (g * NB + i, 0)),
            pl.BlockSpec((BN, CG), lambda i, g: (g * NB + i, 0)),
        ],
        out_shape=[
            jax.ShapeDtypeStruct((G * N, CG), jnp.float32),
            jax.ShapeDtypeStruct((G * N, CG), jnp.bfloat16),
        ],
        compiler_params=pltpu.CompilerParams(
            dimension_semantics=("arbitrary", "arbitrary")),
    )(xf, wblocks, wperm, degT)


def _agg_body(ei_hbm, ew_hbm, tab_hbm, agg_hbm,
              src_v, dst_v, ew_v, gidx_a, didx_a, gidx_b, didx_b,
              bf_a, bf_b, rows_a, rows_b, acc_sh,
              gsem_a, gsem_b, ssem_a, ssem_b):
    c = lax.axis_index("c")
    s = lax.axis_index("s")

    def build_idx(gidx, didx, off, g):
        for q in range(K // 16):
            gidx[pl.ds(q * 16, 16)] = src_v[pl.ds(off + q * 16, 16)] + g * N
            didx[pl.ds(q * 16, 16)] = dst_v[pl.ds(off + q * 16, 16)]

    def scale_rows(bf, rows, off):
        def row(r, _):
            scale = plsc.load_gather(
                ew_v, [jnp.full((16,), off + r, dtype=jnp.int32)])
            for j in range(CG // 32):
                v = plsc.bitcast(bf[r, pl.ds(j * 16, 16)], jnp.bfloat16)
                a, b = plsc.unpack(v, format=plsc.PackFormat.INTERLEAVED)
                rows[r, pl.ds(j * 32, 16)] = a * scale
                rows[r, pl.ds(j * 32 + 16, 16)] = b * scale
            return 0

        lax.fori_loop(0, K, row, 0)

    def gather(gidx, bf, sem):
        pltpu.async_copy(tab_hbm.at[gidx], bf, sem)

    def wait_gather(gidx, bf, sem):
        pltpu.make_async_copy(tab_hbm.at[gidx], bf, sem).wait()

    def scatter(rows, didx, sem):
        pltpu.async_copy(rows, acc_sh.at[didx], sem, add=True)

    def wait_scatter(rows, didx, sem):
        pltpu.make_async_copy(rows, acc_sh.at[didx], sem).wait()

    for gl in range(GPC):
        g = c * GPC + gl

        def zero_rows(i, _):
            for j in range(CG // 16):
                rows_a[i, pl.ds(j * 16, 16)] = jnp.zeros((16,), jnp.float32)
            return 0

        lax.fori_loop(0, K, zero_rows, 0)
        for z in range(ROWS_PT // K):
            pltpu.sync_copy(rows_a, acc_sh.at[pl.ds(s * ROWS_PT + z * K, K)])
        plsc.subcore_barrier()

        def superchunk(scj, _):
            ebase = s * EPT_C + scj * SCE
            pltpu.sync_copy(ei_hbm.at[pl.ds(ebase, SCE)], src_v)
            pltpu.sync_copy(ei_hbm.at[pl.ds(E + ebase, SCE)], dst_v)
            pltpu.sync_copy(ew_hbm.at[pl.ds(ebase, SCE)], ew_v)

            build_idx(gidx_a, didx_a, 0, g)
            gather(gidx_a, bf_a, gsem_a)

            def pair(i, _):
                oa = 2 * i * K
                ob = oa + K
                oa2 = ob + K

                @pl.when(i > 0)
                def _():
                    wait_scatter(rows_b, didx_b, ssem_b)

                build_idx(gidx_b, didx_b, ob, g)
                gather(gidx_b, bf_b, gsem_b)

                wait_gather(gidx_a, bf_a, gsem_a)
                scale_rows(bf_a, rows_a, oa)
                scatter(rows_a, didx_a, ssem_a)

                wait_gather(gidx_b, bf_b, gsem_b)
                scale_rows(bf_b, rows_b, ob)
                scatter(rows_b, didx_b, ssem_b)

                wait_scatter(rows_a, didx_a, ssem_a)
                build_idx(gidx_a, didx_a, oa2, g)
                gather(gidx_a, bf_a, gsem_a)
                return 0

            lax.fori_loop(0, (NBURST - 1) // 2, pair, 0)

            wait_scatter(rows_b, didx_b, ssem_b)
            wait_gather(gidx_a, bf_a, gsem_a)
            scale_rows(bf_a, rows_a, (NBURST - 1) * K)
            pltpu.sync_copy(rows_a, acc_sh.at[didx_a], add=True)
            return 0

        lax.fori_loop(0, NSC, superchunk, 0)
        plsc.subcore_barrier()

        @pl.when(s < NS - 1)
        def _():
            pltpu.sync_copy(acc_sh.at[pl.ds(s * ROWS_PT, ROWS_PT)],
                            agg_hbm.at[pl.ds(g * N + s * ROWS_PT, ROWS_PT)])

        @pl.when(s == NS - 1)
        def _():
            last = N - (NS - 1) * ROWS_PT
            pltpu.sync_copy(acc_sh.at[pl.ds((NS - 1) * ROWS_PT, last)],
                            agg_hbm.at[pl.ds(g * N + (NS - 1) * ROWS_PT, last)])

        plsc.subcore_barrier()


_agg_call = pl.kernel(
    _agg_body,
    out_type=jax.ShapeDtypeStruct((G * N, CG), jnp.float32),
    mesh=_MESH,
    compiler_params=pltpu.CompilerParams(
        needs_layout_passes=False, use_tc_tiling_on_sc=False),
    scratch_types=[
        pltpu.VMEM((SCE,), jnp.int32),
        pltpu.VMEM((SCE,), jnp.int32),
        pltpu.VMEM((SCE,), jnp.float32),
        pltpu.VMEM((K,), jnp.int32),
        pltpu.VMEM((K,), jnp.int32),
        pltpu.VMEM((K,), jnp.int32),
        pltpu.VMEM((K,), jnp.int32),
        pltpu.VMEM((K, CG // 2), jnp.int32),
        pltpu.VMEM((K, CG // 2), jnp.int32),
        pltpu.VMEM((K, CG), jnp.float32),
        pltpu.VMEM((K, CG), jnp.float32),
        pltpu.VMEM_SHARED((NP, CG), jnp.float32),
        pltpu.SemaphoreType.DMA,
        pltpu.SemaphoreType.DMA,
        pltpu.SemaphoreType.DMA,
        pltpu.SemaphoreType.DMA,
    ],
)


def _head_body(agg_ref, tab_ref, deg_ref, bias_ref, pv_ref, lw_ref, lb_ref,
               out_ref):
    dd = deg_ref[...]
    dinv = lax.rsqrt(1.0 + jnp.sum(dd, axis=1, keepdims=True))
    bias = bias_ref[...]
    hsum = jnp.zeros((BN, OUT), jnp.float32)
    for g in range(G):
        S = (agg_ref[g] + tab_ref[g]) * dinv + bias
        Zi = jax.nn.sigmoid(S[:, 0:HCG])
        Hh = jnp.tanh(S[:, HCG:CG])
        Wg = (1.0 - Zi) * Hh * pv_ref[g]
        hsum = hsum + Wg[:, 0:OUT] + Wg[:, OUT:HCG]
    out_ref[...] = (
        jnp.dot(jnp.maximum(hsum, 0.0), lw_ref[...],
                preferred_element_type=jnp.float32) + lb_ref[...])


def _head_call(agg6, tab6, degT, bias128, pvec, linW, linb2):
    return pl.pallas_call(
        _head_body,
        grid=(NB,),
        in_specs=[
            pl.BlockSpec((G, BN, CG), lambda i: (0, i, 0)),
            pl.BlockSpec((G, BN, CG), lambda i: (0, i, 0)),
            pl.BlockSpec((BN, NW), lambda i: (i, 0)),
            pl.BlockSpec((1, CG), lambda i: (0, 0)),
            pl.BlockSpec((G, HCG), lambda i: (0, 0)),
            pl.BlockSpec((OUT, T), lambda i: (0, 0)),
            pl.BlockSpec((1, T), lambda i: (0, 0)),
        ],
        out_specs=pl.BlockSpec((BN, T), lambda i: (i, 0)),
        out_shape=jax.ShapeDtypeStruct((N, T), jnp.float32),
    )(agg6, tab6, degT, bias128, pvec, linW, linb2)


def kernel(x, edge_index, edge_weight, attention, Wz, bz, lzW, lzb, Wr, br,
           lrW, lrb, Wh, bh, lhW, lhb, linW, linb):
    ei = edge_index.astype(jnp.int32).reshape(2 * E)
    ew = edge_weight.astype(jnp.float32)

    Wz2 = Wz @ lzW[:OUT]
    Wh2 = Wh @ lhW[:OUT]
    cz = bz @ lzW[:OUT] + lzb
    ch = bh @ lhW[:OUT] + lhb
    probs = jax.nn.softmax(attention)

    wstack = jnp.stack([Wz2, Wh2], axis=1)
    a = jnp.einsum("ut,fsk->futsk", jnp.eye(T, dtype=jnp.float32), wstack)
    wblocks = (a.reshape(F_IN * T, G, TPG, 2, OUT)
                .transpose(1, 0, 3, 2, 4)
                .reshape(G, F_IN * T, CG))
    bias128 = jnp.concatenate(
        [jnp.tile(cz, TPG), jnp.tile(ch, TPG)]).reshape(1, CG)
    pvec = jnp.repeat(probs, OUT).reshape(G, HCG)

    q = jnp.arange(CG)
    blk, r = q // 32, q % 32
    colmap = blk * 32 + jnp.where(r % 2 == 0, r // 2, 16 + r // 2)
    wperm = wblocks[:, :, colmap]

    degp = _deg_call(ei, ew)
    degT = degp.T
    tab, tab_bf = _dense_call(
        x.reshape(N, F_IN * T), wblocks, wperm, degT)
    tab32 = jax.lax.bitcast_convert_type(
        tab_bf.reshape(G * N, CG // 2, 2), jnp.int32)
    agg = _agg_call(ei, ew, tab32)
    out = _head_call(agg.reshape(G, N, CG), tab.reshape(G, N, CG), degT,
                     bias128, pvec, linW, linb.reshape(1, T))
    return out


# speedup vs baseline: 1.6711x; 1.6711x over previous
import jax
import jax.numpy as jnp
from jax import lax
from jax.experimental import pallas as pl
from jax.experimental.pallas import tpu as pltpu
from jax.experimental.pallas import tpu_sc as plsc

N = 10000
E = 320000
F_IN = 128
OUT = 32
T = 12
G = 6
TPG = T // G
CG = 2 * OUT * TPG
HCG = CG // 2
NC = 2
NS = 16
NW = NC * NS
GPC = G // NC
EPT_A = E // NW
EPT_C = E // NS
SCE = 2000
NSC = EPT_C // SCE
K = 80
NBURST = SCE // K
NP = 10240
ROWS_PT = NP // NS
BN = 400
NB = N // BN

_MESH = plsc.VectorSubcoreMesh(
    core_axis_name="c", subcore_axis_name="s", num_cores=NC, num_subcores=NS)


def _deg_body(ei_hbm, ew_hbm, out_hbm, dst_v, ew_v, hist_v):
    c = lax.axis_index("c")
    s = lax.axis_index("s")
    wid = c * NS + s

    def zero_hist(i, _):
        hist_v[pl.ds(i * 16, 16)] = jnp.zeros((16,), jnp.float32)
        return 0

    lax.fori_loop(0, N // 16, zero_hist, 0)

    pltpu.sync_copy(ei_hbm.at[pl.ds(E + wid * EPT_A, EPT_A)], dst_v)
    pltpu.sync_copy(ew_hbm.at[pl.ds(wid * EPT_A, EPT_A)], ew_v)

    def accum(i, _):
        idx = dst_v[pl.ds(i * 16, 16)]
        w = ew_v[pl.ds(i * 16, 16)]
        plsc.addupdate_scatter(hist_v, [idx], w)
        return 0

    lax.fori_loop(0, EPT_A // 16, accum, 0)

    pltpu.sync_copy(hist_v, out_hbm.at[wid])


_deg_call = pl.kernel(
    _deg_body,
    out_type=jax.ShapeDtypeStruct((NW, N), jnp.float32),
    mesh=_MESH,
    compiler_params=pltpu.CompilerParams(needs_layout_passes=False),
    scratch_types=[
        pltpu.VMEM((EPT_A,), jnp.int32),
        pltpu.VMEM((EPT_A,), jnp.float32),
        pltpu.VMEM((N,), jnp.float32),
    ],
)


def _dense_body(x_ref, w_ref, deg_ref, out_ref):
    acts = jnp.dot(x_ref[...], w_ref[0], preferred_element_type=jnp.float32)
    dd = deg_ref[...]
    dinv = lax.rsqrt(1.0 + jnp.sum(dd, axis=1, keepdims=True))
    out_ref[...] = acts * dinv


def _dense_call(xf, wblocks, degT):
    return pl.pallas_call(
        _dense_body,
        grid=(NB, G),
        in_specs=[
            pl.BlockSpec((BN, F_IN * T), lambda i, g: (i, 0)),
            pl.BlockSpec((1, F_IN * T, CG), lambda i, g: (g, 0, 0)),
            pl.BlockSpec((BN, NW), lambda i, g: (i, 0)),
        ],
        out_specs=pl.BlockSpec((BN, CG), lambda i, g: (g * NB + i, 0)),
        out_shape=jax.ShapeDtypeStruct((G * N, CG), jnp.float32),
        compiler_params=pltpu.CompilerParams(
            dimension_semantics=("arbitrary", "arbitrary")),
    )(xf, wblocks, degT)


NBUF = 4
NQUAD = (NBURST - 1) // NBUF


def _agg_body(ei_hbm, ew_hbm, tab_hbm, agg_hbm,
              src_v, dst_v, ew_v, gidx, didx, rows, acc_sh, gsem, ssem):
    c = lax.axis_index("c")
    s = lax.axis_index("s")

    def build_idx(b, off, g):
        for q in range(K // 16):
            gidx[b][pl.ds(q * 16, 16)] = src_v[pl.ds(off + q * 16, 16)] + g * N
            didx[b][pl.ds(q * 16, 16)] = dst_v[pl.ds(off + q * 16, 16)]

    def scale_rows(b, off):
        def row(r, _):
            scale = plsc.load_gather(
                ew_v, [jnp.full((16,), off + r, dtype=jnp.int32)])
            for j in range(CG // 16):
                rows[b][r, pl.ds(j * 16, 16)] = (
                    rows[b][r, pl.ds(j * 16, 16)] * scale)
            return 0

        lax.fori_loop(0, K, row, 0)

    def gather(b):
        pltpu.async_copy(tab_hbm.at[gidx[b]], rows[b], gsem[b])

    def wait_gather(b):
        pltpu.make_async_copy(tab_hbm.at[gidx[b]], rows[b], gsem[b]).wait()

    def scatter(b):
        pltpu.async_copy(rows[b], acc_sh.at[didx[b]], ssem[b], add=True)

    def wait_scatter(b):
        pltpu.make_async_copy(rows[b], acc_sh.at[didx[b]], ssem[b]).wait()

    for gl in range(GPC):
        g = c * GPC + gl

        def zero_rows(i, _):
            for j in range(CG // 16):
                rows[0][i, pl.ds(j * 16, 16)] = jnp.zeros((16,), jnp.float32)
            return 0

        lax.fori_loop(0, K, zero_rows, 0)
        for z in range(ROWS_PT // K):
            pltpu.sync_copy(rows[0], acc_sh.at[pl.ds(s * ROWS_PT + z * K, K)])
        plsc.subcore_barrier()

        def superchunk(scj, _):
            ebase = s * EPT_C + scj * SCE
            pltpu.sync_copy(ei_hbm.at[pl.ds(ebase, SCE)], src_v)
            pltpu.sync_copy(ei_hbm.at[pl.ds(E + ebase, SCE)], dst_v)
            pltpu.sync_copy(ew_hbm.at[pl.ds(ebase, SCE)], ew_v)

            build_idx(0, 0, g)
            gather(0)

            def quad(i, _):
                base = NBUF * i * K
                for b in range(1, NBUF):

                    @pl.when(i > 0)
                    def _(b=b):
                        wait_scatter(b)

                    build_idx(b, base + b * K, g)
                    gather(b)
                for b in range(NBUF):
                    wait_gather(b)
                    scale_rows(b, base + b * K)
                    scatter(b)
                wait_scatter(0)
                build_idx(0, base + NBUF * K, g)
                gather(0)
                return 0

            lax.fori_loop(0, NQUAD, quad, 0)

            for b in range(1, NBUF):
                wait_scatter(b)
            wait_gather(0)
            scale_rows(0, (NBURST - 1) * K)
            pltpu.sync_copy(rows[0], acc_sh.at[didx[0]], add=True)
            return 0

        lax.fori_loop(0, NSC, superchunk, 0)
        plsc.subcore_barrier()

        @pl.when(s < NS - 1)
        def _():
            pltpu.sync_copy(acc_sh.at[pl.ds(s * ROWS_PT, ROWS_PT)],
                            agg_hbm.at[pl.ds(g * N + s * ROWS_PT, ROWS_PT)])

        @pl.when(s == NS - 1)
        def _():
            last = N - (NS - 1) * ROWS_PT
            pltpu.sync_copy(acc_sh.at[pl.ds((NS - 1) * ROWS_PT, last)],
                            agg_hbm.at[pl.ds(g * N + (NS - 1) * ROWS_PT, last)])

        plsc.subcore_barrier()


_agg_call = pl.kernel(
    _agg_body,
    out_type=jax.ShapeDtypeStruct((G * N, CG), jnp.float32),
    mesh=_MESH,
    compiler_params=pltpu.CompilerParams(needs_layout_passes=False),
    scratch_types=[
        pltpu.VMEM((SCE,), jnp.int32),
        pltpu.VMEM((SCE,), jnp.int32),
        pltpu.VMEM((SCE,), jnp.float32),
        [pltpu.VMEM((K,), jnp.int32) for _ in range(NBUF)],
        [pltpu.VMEM((K,), jnp.int32) for _ in range(NBUF)],
        [pltpu.VMEM((K, CG), jnp.float32) for _ in range(NBUF)],
        pltpu.VMEM_SHARED((NP, CG), jnp.float32),
        [pltpu.SemaphoreType.DMA for _ in range(NBUF)],
        [pltpu.SemaphoreType.DMA for _ in range(NBUF)],
    ],
)


def _head_body(agg_ref, tab_ref, deg_ref, bias_ref, pv_ref, lw_ref, lb_ref,
               out_ref):
    dd = deg_ref[...]
    dinv = lax.rsqrt(1.0 + jnp.sum(dd, axis=1, keepdims=True))
    bias = bias_ref[...]
    hsum = jnp.zeros((BN, OUT), jnp.float32)
    for g in range(G):
        S = (agg_ref[g] + tab_ref[g]) * dinv + bias
        Zi = jax.nn.sigmoid(S[:, 0:HCG])
        Hh = jnp.tanh(S[:, HCG:CG])
        Wg = (1.0 - Zi) * Hh * pv_ref[g]
        hsum = hsum + Wg[:, 0:OUT] + Wg[:, OUT:HCG]
    out_ref[...] = (
        jnp.dot(jnp.maximum(hsum, 0.0), lw_ref[...],
                preferred_element_type=jnp.float32) + lb_ref[...])


def _head_call(agg6, tab6, degT, bias128, pvec, linW, linb2):
    return pl.pallas_call(
        _head_body,
        grid=(NB,),
        in_specs=[
            pl.BlockSpec((G, BN, CG), lambda i: (0, i, 0)),
            pl.BlockSpec((G, BN, CG), lambda i: (0, i, 0)),
            pl.BlockSpec((BN, NW), lambda i: (i, 0)),
            pl.BlockSpec((1, CG), lambda i: (0, 0)),
            pl.BlockSpec((G, HCG), lambda i: (0, 0)),
            pl.BlockSpec((OUT, T), lambda i: (0, 0)),
            pl.BlockSpec((1, T), lambda i: (0, 0)),
        ],
        out_specs=pl.BlockSpec((BN, T), lambda i: (i, 0)),
        out_shape=jax.ShapeDtypeStruct((N, T), jnp.float32),
    )(agg6, tab6, degT, bias128, pvec, linW, linb2)


def kernel(x, edge_index, edge_weight, attention, Wz, bz, lzW, lzb, Wr, br,
           lrW, lrb, Wh, bh, lhW, lhb, linW, linb):
    ei = edge_index.astype(jnp.int32).reshape(2 * E)
    ew = edge_weight.astype(jnp.float32)

    Wz2 = Wz @ lzW[:OUT]
    Wh2 = Wh @ lhW[:OUT]
    cz = bz @ lzW[:OUT] + lzb
    ch = bh @ lhW[:OUT] + lhb
    probs = jax.nn.softmax(attention)

    wstack = jnp.stack([Wz2, Wh2], axis=1)
    a = jnp.einsum("ut,fsk->futsk", jnp.eye(T, dtype=jnp.float32), wstack)
    wblocks = (a.reshape(F_IN * T, G, TPG, 2, OUT)
                .transpose(1, 0, 3, 2, 4)
                .reshape(G, F_IN * T, CG))
    bias128 = jnp.concatenate(
        [jnp.tile(cz, TPG), jnp.tile(ch, TPG)]).reshape(1, CG)
    pvec = jnp.repeat(probs, OUT).reshape(G, HCG)

    degp = _deg_call(ei, ew)
    degT = degp.T
    tab = _dense_call(x.reshape(N, F_IN * T), wblocks, degT)
    agg = _agg_call(ei, ew, tab)
    out = _head_call(agg.reshape(G, N, CG), tab.reshape(G, N, CG), degT,
                     bias128, pvec, linW, linb.reshape(1, T))
    return out


# speedup vs baseline: 1.6975x; 1.0158x over previous
import jax
import jax.numpy as jnp
from jax import lax
from jax.experimental import pallas as pl
from jax.experimental.pallas import tpu as pltpu
from jax.experimental.pallas import tpu_sc as plsc

N = 10000
E = 320000
F_IN = 128
OUT = 32
T = 12
G = 6
TPG = T // G
CG = 2 * OUT * TPG
HCG = CG // 2
NC = 2
NS = 16
NW = NC * NS
GPC = G // NC
EPT_A = E // NW
EPT_C = E // NS
SCE = 2000
NSC = EPT_C // SCE
K = 80
NBURST = SCE // K
NP = 10240
ROWS_PT = NP // NS
BN = 400
NB = N // BN

_MESH = plsc.VectorSubcoreMesh(
    core_axis_name="c", subcore_axis_name="s", num_cores=NC, num_subcores=NS)


def _deg_body(ei_hbm, ew_hbm, dinv_hbm, dst_v, ew_v, hist_v, red_v, out_v,
              acc_sh):
    c = lax.axis_index("c")
    s = lax.axis_index("s")

    @pl.when(c == 0)
    def _():
        def zero_hist(i, _):
            hist_v[pl.ds(i * 16, 16)] = jnp.zeros((16,), jnp.float32)
            return 0

        lax.fori_loop(0, NP // 16, zero_hist, 0)

        pltpu.sync_copy(ei_hbm.at[pl.ds(E + s * EPT_C, EPT_C)], dst_v)
        pltpu.sync_copy(ew_hbm.at[pl.ds(s * EPT_C, EPT_C)], ew_v)

        def accum(i, _):
            idx = dst_v[pl.ds(i * 16, 16)]
            w = ew_v[pl.ds(i * 16, 16)]
            plsc.addupdate_scatter(hist_v, [idx], w)
            return 0

        lax.fori_loop(0, EPT_C // 16, accum, 0)

        pltpu.sync_copy(hist_v, acc_sh.at[s])
        plsc.subcore_barrier()

        pltpu.sync_copy(acc_sh.at[:, pl.ds(s * ROWS_PT, ROWS_PT)], red_v)

        def red(i, _):
            t = jnp.full((16,), 1.0, jnp.float32)
            for r in range(NS):
                t = t + red_v[r, pl.ds(i * 16, 16)]
            xh = t * 0.5
            ii = plsc.bitcast(t, jnp.int32)
            ii = 0x5F3759DF - lax.shift_right_logical(ii, 1)
            y = plsc.bitcast(ii, jnp.float32)
            y = y * (1.5 - xh * y * y)
            y = y * (1.5 - xh * y * y)
            y = y * (1.5 - xh * y * y)
            out_v[pl.ds(i * 16, 16)] = y
            return 0

        lax.fori_loop(0, ROWS_PT // 16, red, 0)
        pltpu.sync_copy(out_v, dinv_hbm.at[pl.ds(s * ROWS_PT, ROWS_PT)])


_deg_call = pl.kernel(
    _deg_body,
    out_type=jax.ShapeDtypeStruct((NP,), jnp.float32),
    mesh=_MESH,
    compiler_params=pltpu.CompilerParams(needs_layout_passes=False),
    scratch_types=[
        pltpu.VMEM((EPT_C,), jnp.int32),
        pltpu.VMEM((EPT_C,), jnp.float32),
        pltpu.VMEM((NP,), jnp.float32),
        pltpu.VMEM((NS, NP // NS), jnp.float32),
        pltpu.VMEM((NP // NS,), jnp.float32),
        pltpu.VMEM_SHARED((NS, NP), jnp.float32),
    ],
)


def _dense_body(x_ref, w_ref, dinv_ref, out_ref):
    xb = x_ref[...].astype(jnp.bfloat16)
    acts = jnp.dot(xb, w_ref[0], preferred_element_type=jnp.float32)
    out_ref[...] = acts * dinv_ref[...]


def _dense_call(xf, wblocks, dinv2):
    return pl.pallas_call(
        _dense_body,
        grid=(NB, G),
        in_specs=[
            pl.BlockSpec((BN, F_IN * T), lambda i, g: (i, 0)),
            pl.BlockSpec((1, F_IN * T, CG), lambda i, g: (g, 0, 0)),
            pl.BlockSpec((BN, 1), lambda i, g: (i, 0)),
        ],
        out_specs=pl.BlockSpec((BN, CG), lambda i, g: (g * NB + i, 0)),
        out_shape=jax.ShapeDtypeStruct((G * N, CG), jnp.float32),
        compiler_params=pltpu.CompilerParams(
            dimension_semantics=("arbitrary", "arbitrary")),
    )(xf, wblocks, dinv2)


NBUF = 4
NQUAD = (NBURST - 1) // NBUF


def _agg_body(ei_hbm, ew_hbm, tab_hbm, agg_hbm,
              src_v, dst_v, ew_v, gidx, didx, rows, acc_sh, gsem, ssem):
    c = lax.axis_index("c")
    s = lax.axis_index("s")

    def build_idx(b, off, g):
        for q in range(K // 16):
            gidx[b][pl.ds(q * 16, 16)] = src_v[pl.ds(off + q * 16, 16)] + g * N
            didx[b][pl.ds(q * 16, 16)] = dst_v[pl.ds(off + q * 16, 16)]

    def scale_rows(b, off):
        def row(r, _):
            scale = plsc.load_gather(
                ew_v, [jnp.full((16,), off + r, dtype=jnp.int32)])
            for j in range(CG // 16):
                rows[b][r, pl.ds(j * 16, 16)] = (
                    rows[b][r, pl.ds(j * 16, 16)] * scale)
            return 0

        lax.fori_loop(0, K, row, 0)

    def gather(b):
        pltpu.async_copy(tab_hbm.at[gidx[b]], rows[b], gsem[b])

    def wait_gather(b):
        pltpu.make_async_copy(tab_hbm.at[gidx[b]], rows[b], gsem[b]).wait()

    def scatter(b):
        pltpu.async_copy(rows[b], acc_sh.at[didx[b]], ssem[b], add=True)

    def wait_scatter(b):
        pltpu.make_async_copy(rows[b], acc_sh.at[didx[b]], ssem[b]).wait()

    for gl in range(GPC):
        g = c * GPC + gl

        def zero_rows(i, _):
            for j in range(CG // 16):
                rows[0][i, pl.ds(j * 16, 16)] = jnp.zeros((16,), jnp.float32)
            return 0

        lax.fori_loop(0, K, zero_rows, 0)
        for z in range(ROWS_PT // K):
            pltpu.sync_copy(rows[0], acc_sh.at[pl.ds(s * ROWS_PT + z * K, K)])
        plsc.subcore_barrier()

        def superchunk(scj, _):
            ebase = s * EPT_C + scj * SCE
            pltpu.sync_copy(ei_hbm.at[pl.ds(ebase, SCE)], src_v)
            pltpu.sync_copy(ei_hbm.at[pl.ds(E + ebase, SCE)], dst_v)
            pltpu.sync_copy(ew_hbm.at[pl.ds(ebase, SCE)], ew_v)

            build_idx(0, 0, g)
            gather(0)

            def quad(i, _):
                base = NBUF * i * K
                for b in range(1, NBUF):

                    @pl.when(i > 0)
                    def _(b=b):
                        wait_scatter(b)

                    build_idx(b, base + b * K, g)
                    gather(b)
                for b in range(NBUF):
                    wait_gather(b)
                    scale_rows(b, base + b * K)
                    scatter(b)
                wait_scatter(0)
                build_idx(0, base + NBUF * K, g)
                gather(0)
                return 0

            lax.fori_loop(0, NQUAD, quad, 0)

            for b in range(1, NBUF):
                wait_scatter(b)
            wait_gather(0)
            scale_rows(0, (NBURST - 1) * K)
            pltpu.sync_copy(rows[0], acc_sh.at[didx[0]], add=True)
            return 0

        lax.fori_loop(0, NSC, superchunk, 0)
        plsc.subcore_barrier()

        @pl.when(s < NS - 1)
        def _():
            pltpu.sync_copy(acc_sh.at[pl.ds(s * ROWS_PT, ROWS_PT)],
                            agg_hbm.at[pl.ds(g * N + s * ROWS_PT, ROWS_PT)])

        @pl.when(s == NS - 1)
        def _():
            last = N - (NS - 1) * ROWS_PT
            pltpu.sync_copy(acc_sh.at[pl.ds((NS - 1) * ROWS_PT, last)],
                            agg_hbm.at[pl.ds(g * N + (NS - 1) * ROWS_PT, last)])

        plsc.subcore_barrier()


_agg_call = pl.kernel(
    _agg_body,
    out_type=jax.ShapeDtypeStruct((G * N, CG), jnp.float32),
    mesh=_MESH,
    compiler_params=pltpu.CompilerParams(needs_layout_passes=False),
    scratch_types=[
        pltpu.VMEM((SCE,), jnp.int32),
        pltpu.VMEM((SCE,), jnp.int32),
        pltpu.VMEM((SCE,), jnp.float32),
        [pltpu.VMEM((K,), jnp.int32) for _ in range(NBUF)],
        [pltpu.VMEM((K,), jnp.int32) for _ in range(NBUF)],
        [pltpu.VMEM((K, CG), jnp.float32) for _ in range(NBUF)],
        pltpu.VMEM_SHARED((NP, CG), jnp.float32),
        [pltpu.SemaphoreType.DMA for _ in range(NBUF)],
        [pltpu.SemaphoreType.DMA for _ in range(NBUF)],
    ],
)


def _head_body(agg_ref, tab_ref, dinv_ref, bias_ref, pv_ref, lw_ref, lb_ref,
               out_ref):
    dinv = dinv_ref[...]
    bias = bias_ref[...]
    hsum = jnp.zeros((BN, OUT), jnp.float32)
    for g in range(G):
        S = (agg_ref[g] + tab_ref[g]) * dinv + bias
        Zi = jax.nn.sigmoid(S[:, 0:HCG])
        Hh = jnp.tanh(S[:, HCG:CG])
        Wg = (1.0 - Zi) * Hh * pv_ref[g]
        hsum = hsum + Wg[:, 0:OUT] + Wg[:, OUT:HCG]
    out_ref[...] = (
        jnp.dot(jnp.maximum(hsum, 0.0), lw_ref[...],
                preferred_element_type=jnp.float32) + lb_ref[...])


def _head_call(agg6, tab6, dinv2, bias128, pvec, linW, linb2):
    return pl.pallas_call(
        _head_body,
        grid=(NB,),
        in_specs=[
            pl.BlockSpec((G, BN, CG), lambda i: (0, i, 0)),
            pl.BlockSpec((G, BN, CG), lambda i: (0, i, 0)),
            pl.BlockSpec((BN, 1), lambda i: (i, 0)),
            pl.BlockSpec((1, CG), lambda i: (0, 0)),
            pl.BlockSpec((G, HCG), lambda i: (0, 0)),
            pl.BlockSpec((OUT, T), lambda i: (0, 0)),
            pl.BlockSpec((1, T), lambda i: (0, 0)),
        ],
        out_specs=pl.BlockSpec((BN, T), lambda i: (i, 0)),
        out_shape=jax.ShapeDtypeStruct((N, T), jnp.float32),
    )(agg6, tab6, dinv2, bias128, pvec, linW, linb2)


def kernel(x, edge_index, edge_weight, attention, Wz, bz, lzW, lzb, Wr, br,
           lrW, lrb, Wh, bh, lhW, lhb, linW, linb):
    ei = edge_index.astype(jnp.int32).reshape(2 * E)
    ew = edge_weight.astype(jnp.float32)

    Wz2 = Wz @ lzW[:OUT]
    Wh2 = Wh @ lhW[:OUT]
    cz = bz @ lzW[:OUT] + lzb
    ch = bh @ lhW[:OUT] + lhb
    probs = jax.nn.softmax(attention)

    wstack = jnp.stack([Wz2, Wh2], axis=1)
    a = jnp.einsum("ut,fsk->futsk", jnp.eye(T, dtype=jnp.float32), wstack)
    wblocks = (a.reshape(F_IN * T, G, TPG, 2, OUT)
                .transpose(1, 0, 3, 2, 4)
                .reshape(G, F_IN * T, CG))
    bias128 = jnp.concatenate(
        [jnp.tile(cz, TPG), jnp.tile(ch, TPG)]).reshape(1, CG)
    pvec = jnp.repeat(probs, OUT).reshape(G, HCG)

    dinv2 = _deg_call(ei, ew).reshape(NP, 1)
    tab = _dense_call(x.reshape(N, F_IN * T),
                      wblocks.astype(jnp.bfloat16), dinv2)
    agg = _agg_call(ei, ew, tab)
    out = _head_call(agg.reshape(G, N, CG), tab.reshape(G, N, CG), dinv2,
                     bias128, pvec, linW, linb.reshape(1, T))
    return out
